# baseline (speedup 1.0000x reference)
import jax
import jax.numpy as jnp
from jax import lax
from jax.experimental import pallas as pl
from jax.experimental.pallas import tpu as pltpu, tpu_sc as plsc

N_ROWS = 1000001
K = 64
T = 50
B = 16384

NC = 2
NS = 16
NW = NC * NS
CHUNK = B // NW
G = 16
NG = CHUNK // G
KV = K // 16


def _body(users_hbm, items_hbm, seq_hbm, vui_hbm, viu_hbm, vil_hbm,
          vli_hbm, out_hbm,
          users_s, items_s, seq_s, idx_v, seqv_v, stage_v, u_v, iu_v, il_v,
          out_v, sem_hist, sem_rows):
    wid = lax.axis_index("s") * NC + lax.axis_index("c")
    base = wid * CHUNK

    pltpu.sync_copy(users_hbm.at[pl.ds(base, CHUNK)], idx_v)

    @pl.loop(0, CHUNK // 16)
    def _stage_u(q):
        v = idx_v[pl.ds(pl.multiple_of(q * 16, 16), 16)]
        for j in range(16):
            users_s[q * 16 + j] = jax.lax.index_in_dim(v, j, 0, False)

    pltpu.sync_copy(items_hbm.at[pl.ds(base, CHUNK)], idx_v)

    @pl.loop(0, CHUNK // 16)
    def _stage_i(q):
        v = idx_v[pl.ds(pl.multiple_of(q * 16, 16), 16)]
        for j in range(16):
            items_s[q * 16 + j] = jax.lax.index_in_dim(v, j, 0, False)

    @pl.loop(0, NG)
    def _grp(g):
        off = pl.multiple_of(g * G, G)

        pltpu.sync_copy(seq_hbm.at[pl.ds((base + off) * T, G * T)], seqv_v)

        @pl.loop(0, (G * T) // 16)
        def _stage_s(q):
            v = seqv_v[pl.ds(pl.multiple_of(q * 16, 16), 16)]
            for j in range(16):
                seq_s[q * 16 + j] = jax.lax.index_in_dim(v, j, 0, False)

        @pl.loop(0, G)
        def _fire(r):
            @pl.loop(0, T)
            def _hist(t):
                s = seq_s[r * T + t]
                pltpu.async_copy(vli_hbm.at[pl.ds(s, 1)],
                                 stage_v.at[pl.ds(r * T + t, 1)], sem_hist)

            u = users_s[off + r]
            i = items_s[off + r]
            pltpu.async_copy(vui_hbm.at[pl.ds(u, 1)], u_v.at[pl.ds(r, 1)],
                             sem_rows)
            pltpu.async_copy(viu_hbm.at[pl.ds(i, 1)], iu_v.at[pl.ds(r, 1)],
                             sem_rows)
            pltpu.async_copy(vil_hbm.at[pl.ds(i, 1)], il_v.at[pl.ds(r, 1)],
                             sem_rows)

        pltpu.make_async_copy(vli_hbm.at[pl.ds(0, G * T)], stage_v,
                              sem_hist).wait()
        pltpu.make_async_copy(vui_hbm.at[pl.ds(0, G)], u_v, sem_rows).wait()
        pltpu.make_async_copy(vui_hbm.at[pl.ds(0, G)], iu_v, sem_rows).wait()
        pltpu.make_async_copy(vui_hbm.at[pl.ds(0, G)], il_v, sem_rows).wait()

        lanes = lax.iota(jnp.int32, 16)
        zero16 = jnp.zeros((16,), jnp.float32)

        @pl.loop(0, G, init_carry=zero16)
        def score_vec(j, sc):
            @pl.loop(0, T, init_carry=jnp.int32(0))
            def cnt(t, c):
                return c + jnp.where(seq_s[j * T + t] != 0, 1, 0)

            inv = 1.0 / jnp.maximum(zero16 + cnt.astype(jnp.float32), 1.0)

            il = [il_v[j, pl.ds(k * 16, 16)] for k in range(KV)]
            s_ui = zero16
            for k in range(KV):
                ks = pl.ds(k * 16, 16)
                s_ui = s_ui + u_v[j, ks] * iu_v[j, ks]

            @pl.loop(0, T, init_carry=tuple(zero16 for _ in range(KV)))
            def s_il(t, carry):
                row = j * T + t
                return tuple(
                    carry[k] + il[k] * stage_v[row, pl.ds(k * 16, 16)]
                    for k in range(KV))

            s_il_tot = s_il[0]
            for k in range(1, KV):
                s_il_tot = s_il_tot + s_il[k]

            onehot = jnp.where(lanes == j, 1.0, 0.0).astype(jnp.float32)
            return (sc + jnp.sum(s_ui) * onehot
                    + (jnp.sum(s_il_tot) * onehot) * inv)

        out_v[pl.ds(off, G)] = score_vec

    pltpu.sync_copy(out_v, out_hbm.at[pl.ds(base, CHUNK)])


@jax.jit
def kernel(users, items, seq_padded, VUI, VIU, VIL, VLI):
    seq_flat = jnp.asarray(seq_padded, jnp.int32).reshape(B * T)
    users = jnp.asarray(users, jnp.int32)
    items = jnp.asarray(items, jnp.int32)

    call = pl.kernel(
        _body,
        out_type=jax.ShapeDtypeStruct((B,), jnp.float32),
        mesh=plsc.VectorSubcoreMesh(core_axis_name="c", subcore_axis_name="s"),
        compiler_params=pltpu.CompilerParams(use_tc_tiling_on_sc=True,
                                             needs_layout_passes=False),
        scratch_types=[
            pltpu.SMEM((CHUNK,), jnp.int32),
            pltpu.SMEM((CHUNK,), jnp.int32),
            pltpu.SMEM((G * T,), jnp.int32),
            pltpu.VMEM((CHUNK,), jnp.int32),
            pltpu.VMEM((G * T,), jnp.int32),
            pltpu.VMEM((G * T, K), jnp.float32),
            pltpu.VMEM((G, K), jnp.float32),
            pltpu.VMEM((G, K), jnp.float32),
            pltpu.VMEM((G, K), jnp.float32),
            pltpu.VMEM((CHUNK,), jnp.float32),
            pltpu.SemaphoreType.DMA,
            pltpu.SemaphoreType.DMA,
        ],
    )
    return call(users, items, seq_flat, VUI, VIU, VIL, VLI)
